# baseline (speedup 1.0000x reference)
import functools

import numpy as np
import jax
import jax.numpy as jnp
from jax import lax
from jax.experimental import pallas as pl
from jax.experimental.pallas import tpu as pltpu
from jax.experimental.pallas import tpu_sc as plsc

_B = 4096
_F = 26
_V = 100000
_D = 64
_BF = _B * _F
_NP1 = _F + 1
_H0 = 1024
_GW = 128
_BB = 256

_IU, _JU = np.triu_indices(_NP1, k=1)
_S3_GATHER = np.full((_NP1 * _NP1,), len(_IU), dtype=np.int32)
_S3_GATHER[_IU * _NP1 + _JU] = np.arange(len(_IU), dtype=np.int32)


_NW = 32
_RPW = _BF // _NW
_CH = 256


def _sc_gather(table, idx):
  mesh = plsc.VectorSubcoreMesh(core_axis_name="c", subcore_axis_name="s")

  n_ch = _RPW // _CH

  @functools.partial(
      pl.kernel,
      out_type=jax.ShapeDtypeStruct((_BF, 2 * _D), jnp.float32),
      mesh=mesh,
      scratch_types=[
          pltpu.VMEM((2, _CH), jnp.int32),
          pltpu.VMEM((2, _CH, 2 * _D), jnp.float32),
          pltpu.SemaphoreType.DMA,
          pltpu.SemaphoreType.DMA,
          pltpu.SemaphoreType.DMA,
      ],
  )
  def k(x_hbm, i_hbm, o_hbm, idx_v, buf, sem_i, sem_g, sem_w):
    wid = lax.axis_index("s") * 2 + lax.axis_index("c")
    base = wid * _RPW
    pltpu.make_async_copy(
        i_hbm.at[0, pl.ds(base, _CH)], idx_v.at[0], sem_i).start()
    for c in range(n_ch):
      p = c & 1
      cbase = base + c * _CH
      pltpu.make_async_copy(
          i_hbm.at[0, pl.ds(cbase, _CH)], idx_v.at[p], sem_i).wait()
      if c + 1 < n_ch:
        pltpu.make_async_copy(
            i_hbm.at[0, pl.ds(cbase + _CH, _CH)],
            idx_v.at[1 - p], sem_i).start()
      if c >= 2:
        pltpu.make_async_copy(
            buf.at[p], o_hbm.at[pl.ds(cbase, _CH), :], sem_w).wait()

      @pl.loop(0, _CH // 16)
      def _(g):
        vec = idx_v[p, pl.ds(g * 16, 16)]
        for j in range(16):
          r = vec[j]
          pltpu.make_async_copy(
              x_hbm.at[pl.ds(r, 1), :],
              buf.at[p].at[pl.ds(g * 16 + j, 1), :], sem_g).start()

      pltpu.make_async_copy(x_hbm.at[pl.ds(0, _CH), :], buf.at[p], sem_g).wait()
      pltpu.make_async_copy(
          buf.at[p], o_hbm.at[pl.ds(cbase, _CH), :], sem_w).start()
    for p in range(2):
      pltpu.make_async_copy(
          buf.at[p], o_hbm.at[pl.ds(base, _CH), :], sem_w).wait()

  return k(table, idx)


def _tc_body(d_ref, e_ref, p_ref, dw0_ref, dw1_ref, dw2_ref, db0_ref, db1_ref,
             db2_ref, w1x_ref, s3_ref, ow1_ref, ow2_ref, ow3_ref, ow4_ref,
             ob0_ref, ob1_ref, ob2_ref, ob3_ref, ob4_ref, o_ref):
  f32 = jnp.float32
  x = d_ref[...]
  x = jnp.maximum(jnp.dot(x, dw0_ref[...], preferred_element_type=f32)
                  + db0_ref[...], 0.0)
  x = jnp.maximum(jnp.dot(x, dw1_ref[...], preferred_element_type=f32)
                  + db1_ref[...], 0.0)
  x = jnp.maximum(jnp.dot(x, dw2_ref[...], preferred_element_type=f32)
                  + db2_ref[...], 0.0)
  e128 = e_ref[...]
  par = p_ref[...]
  emb = e128[:, :, :_D] * (1.0 - par) + e128[:, :, _D:] * par
  c3 = jnp.concatenate([x[:, None, :], emb], axis=1)
  z3 = lax.dot_general(c3, c3, (((2,), (2,)), ((0,), (0,))),
                       preferred_element_type=f32)
  h = jnp.dot(x, w1x_ref[...], preferred_element_type=f32) + ob0_ref[...]
  for i in range(_F):
    h = h + jnp.dot(z3[:, i], s3_ref[i], preferred_element_type=f32)
  h = jnp.maximum(h, 0.0)
  h = jnp.maximum(jnp.dot(h, ow1_ref[...], preferred_element_type=f32)
                  + ob1_ref[...], 0.0)
  h = jnp.maximum(jnp.dot(h, ow2_ref[...], preferred_element_type=f32)
                  + ob2_ref[...], 0.0)
  h = jnp.maximum(jnp.dot(h, ow3_ref[...], preferred_element_type=f32)
                  + ob3_ref[...], 0.0)
  o_ref[...] = (jnp.dot(h, ow4_ref[...], preferred_element_type=f32)
                + ob4_ref[...])


def _tc_forward(dense_p, emb3, par3, dw0p, dw1, dw2, db0, db1, db2, w1x, s3,
                ow1, ow2, ow3, ow4, ob0, ob1, ob2, ob3, ob4):
  full = lambda a: pl.BlockSpec(a.shape, lambda i: (0,) * a.ndim)
  return pl.pallas_call(
      _tc_body,
      grid=(_B // _BB,),
      in_specs=[
          pl.BlockSpec((_BB, 16), lambda i: (i, 0)),
          pl.BlockSpec((_BB, _F, 2 * _D), lambda i: (i, 0, 0)),
          pl.BlockSpec((_BB, _F, 1), lambda i: (i, 0, 0)),
          full(dw0p), full(dw1), full(dw2),
          full(db0), full(db1), full(db2),
          full(w1x), full(s3),
          full(ow1), full(ow2), full(ow3), full(ow4),
          full(ob0), full(ob1), full(ob2), full(ob3), full(ob4),
      ],
      out_specs=pl.BlockSpec((_BB, 1), lambda i: (i, 0)),
      out_shape=jax.ShapeDtypeStruct((_B, 1), jnp.float32),
  )(dense_p, emb3, par3, dw0p, dw1, dw2, db0, db1, db2, w1x, s3,
    ow1, ow2, ow3, ow4, ob0, ob1, ob2, ob3, ob4)


def kernel(dense_features, sparse_features, emb_tables, dw0, db0, dw1, db1,
           dw2, db2, ow0, ob0, ow1, ob1, ow2, ob2, ow3, ob3, ow4, ob4):
  table = emb_tables.reshape(_F * _V // 2, 2 * _D)
  offs = (jnp.arange(_F, dtype=jnp.int32) * _V)[None, :]
  r = sparse_features.astype(jnp.int32) + offs
  idx = (r // 2).reshape(1, _BF)
  emb = _sc_gather(table, idx)
  emb3 = emb.reshape(_B, _F, 2 * _D)
  par3 = (r % 2).astype(jnp.float32).reshape(_B, _F, 1)

  dense_p = jnp.pad(dense_features, ((0, 0), (0, 3)))
  dw0p = jnp.pad(dw0, ((0, 3), (0, 0)))
  w1x = ow0[:_D]
  w2ext = jnp.concatenate(
      [ow0[_D:], jnp.zeros((1, _H0), jnp.float32)], axis=0)
  s3 = w2ext[_S3_GATHER].reshape(_NP1, _NP1, _H0)

  r2 = lambda b: b.reshape(1, -1)
  return _tc_forward(dense_p, emb3, par3, dw0p, dw1, dw2,
                     r2(db0), r2(db1), r2(db2), w1x, s3,
                     ow1, ow2, ow3, ow4,
                     r2(ob0), r2(ob1), r2(ob2), r2(ob3), r2(ob4))


# speedup vs baseline: 2.4955x; 2.4955x over previous
import functools

import numpy as np
import jax
import jax.numpy as jnp
from jax import lax
from jax.experimental import pallas as pl
from jax.experimental.pallas import tpu as pltpu
from jax.experimental.pallas import tpu_sc as plsc

_B = 4096
_F = 26
_V = 100000
_D = 64
_BF = _B * _F
_NP1 = _F + 1
_H0 = 1024
_GW = 128
_BB = 512

_IU, _JU = np.triu_indices(_NP1, k=1)
_S3_GATHER = np.full((_NP1 * _NP1,), len(_IU), dtype=np.int32)
_S3_GATHER[_IU * _NP1 + _JU] = np.arange(len(_IU), dtype=np.int32)


_NW = 32
_RPW = _BF // _NW
_CH = 256


def _sc_gather(table, idx):
  mesh = plsc.VectorSubcoreMesh(core_axis_name="c", subcore_axis_name="s")

  n_ch = _RPW // _CH

  @functools.partial(
      pl.kernel,
      out_type=jax.ShapeDtypeStruct((_BF, _D), jnp.float32),
      mesh=mesh,
      scratch_types=[
          pltpu.VMEM((2, _CH), jnp.int32),
          pltpu.VMEM((2, _CH, _D), jnp.float32),
          pltpu.SemaphoreType.DMA,
          pltpu.SemaphoreType.DMA,
          pltpu.SemaphoreType.DMA,
      ],
  )
  def k(x_hbm, i_hbm, o_hbm, idx_v, buf, sem_i, sem_g, sem_w):
    wid = lax.axis_index("s") * 2 + lax.axis_index("c")
    base = wid * _RPW
    pltpu.make_async_copy(
        i_hbm.at[0, pl.ds(base, _CH)], idx_v.at[0], sem_i).start()
    for c in range(n_ch):
      p = c & 1
      cbase = base + c * _CH
      pltpu.make_async_copy(
          i_hbm.at[0, pl.ds(cbase, _CH)], idx_v.at[p], sem_i).wait()
      if c + 1 < n_ch:
        pltpu.make_async_copy(
            i_hbm.at[0, pl.ds(cbase + _CH, _CH)],
            idx_v.at[1 - p], sem_i).start()
      if c >= 2:
        pltpu.make_async_copy(
            buf.at[p], o_hbm.at[pl.ds(cbase, _CH), :], sem_w).wait()

      @pl.loop(0, _CH // 16)
      def _(g):
        vec = idx_v[p, pl.ds(g * 16, 16)]
        for j in range(16):
          r = vec[j]
          pltpu.make_async_copy(
              x_hbm.at[pl.ds(r, 1), :],
              buf.at[p].at[pl.ds(g * 16 + j, 1), :], sem_g).start()

      pltpu.make_async_copy(x_hbm.at[pl.ds(0, _CH), :], buf.at[p], sem_g).wait()
      pltpu.make_async_copy(
          buf.at[p], o_hbm.at[pl.ds(cbase, _CH), :], sem_w).start()
    for p in range(2):
      pltpu.make_async_copy(
          buf.at[p], o_hbm.at[pl.ds(base, _CH), :], sem_w).wait()

  return k(table, idx)


def _tc_body(d_ref, e_ref, dw0_ref, dw1_ref, dw2_ref, db0_ref, db1_ref,
             db2_ref, w1x_ref, s3_ref, ow1_ref, ow2_ref, ow3_ref, ow4_ref,
             ob0_ref, ob1_ref, ob2_ref, ob3_ref, ob4_ref, o_ref):
  f32 = jnp.float32
  x = d_ref[...]
  x = jnp.maximum(jnp.dot(x, dw0_ref[...], preferred_element_type=f32)
                  + db0_ref[...], 0.0)
  x = jnp.maximum(jnp.dot(x, dw1_ref[...], preferred_element_type=f32)
                  + db1_ref[...], 0.0)
  x = jnp.maximum(jnp.dot(x, dw2_ref[...], preferred_element_type=f32)
                  + db2_ref[...], 0.0)
  c3 = jnp.concatenate([x[:, None, :], e_ref[...]], axis=1)
  z3 = lax.dot_general(c3, c3, (((2,), (2,)), ((0,), (0,))),
                       preferred_element_type=f32)
  h = jnp.dot(x, w1x_ref[...], preferred_element_type=f32) + ob0_ref[...]
  for i in range(_F):
    h = h + jnp.dot(z3[:, i], s3_ref[i], preferred_element_type=f32)
  h = jnp.maximum(h, 0.0)
  h = jnp.maximum(jnp.dot(h, ow1_ref[...], preferred_element_type=f32)
                  + ob1_ref[...], 0.0)
  h = jnp.maximum(jnp.dot(h, ow2_ref[...], preferred_element_type=f32)
                  + ob2_ref[...], 0.0)
  h = jnp.maximum(jnp.dot(h, ow3_ref[...], preferred_element_type=f32)
                  + ob3_ref[...], 0.0)
  o_ref[...] = (jnp.dot(h, ow4_ref[...], preferred_element_type=f32)
                + ob4_ref[...])


def _tc_forward(dense_p, emb3, dw0p, dw1, dw2, db0, db1, db2, w1x, s3,
                ow1, ow2, ow3, ow4, ob0, ob1, ob2, ob3, ob4):
  full = lambda a: pl.BlockSpec(a.shape, lambda i: (0,) * a.ndim)
  return pl.pallas_call(
      _tc_body,
      grid=(_B // _BB,),
      in_specs=[
          pl.BlockSpec((_BB, 16), lambda i: (i, 0)),
          pl.BlockSpec((_BB, _F, _D), lambda i: (i, 0, 0)),
          full(dw0p), full(dw1), full(dw2),
          full(db0), full(db1), full(db2),
          full(w1x), full(s3),
          full(ow1), full(ow2), full(ow3), full(ow4),
          full(ob0), full(ob1), full(ob2), full(ob3), full(ob4),
      ],
      out_specs=pl.BlockSpec((_BB, 1), lambda i: (i, 0)),
      out_shape=jax.ShapeDtypeStruct((_B, 1), jnp.float32),
  )(dense_p, emb3, dw0p, dw1, dw2, db0, db1, db2, w1x, s3,
    ow1, ow2, ow3, ow4, ob0, ob1, ob2, ob3, ob4)


def kernel(dense_features, sparse_features, emb_tables, dw0, db0, dw1, db1,
           dw2, db2, ow0, ob0, ow1, ob1, ow2, ob2, ow3, ob3, ow4, ob4):
  table = emb_tables.reshape(_F * _V, _D)
  offs = (jnp.arange(_F, dtype=jnp.int32) * _V)[None, :]
  idx = (sparse_features.astype(jnp.int32) + offs).reshape(1, _BF)
  emb = _sc_gather(table, idx)
  emb3 = emb.reshape(_B, _F, _D)

  dense_p = jnp.pad(dense_features, ((0, 0), (0, 3)))
  dw0p = jnp.pad(dw0, ((0, 3), (0, 0)))
  w1x = ow0[:_D]
  w2ext = jnp.concatenate(
      [ow0[_D:], jnp.zeros((1, _H0), jnp.float32)], axis=0)
  s3 = w2ext[_S3_GATHER].reshape(_NP1, _NP1, _H0)

  r2 = lambda b: b.reshape(1, -1)
  return _tc_forward(dense_p, emb3, dw0p, dw1, dw2,
                     r2(db0), r2(db1), r2(db2), w1x, s3,
                     ow1, ow2, ow3, ow4,
                     r2(ob0), r2(ob1), r2(ob2), r2(ob3), r2(ob4))
